# baseline (speedup 1.0000x reference)
import functools

import jax
import jax.numpy as jnp
from jax import lax
from jax.experimental import pallas as pl
from jax.experimental.pallas import tpu as pltpu
from jax.experimental.pallas import tpu_sc as plsc

HIDDEN = 768
NUM_CORES = 2
NUM_SUBCORES = 16
NUM_WORKERS = NUM_CORES * NUM_SUBCORES

B_TOTAL = 4 * 8192
B_PER_W = B_TOTAL // NUM_WORKERS
K_INNER = 16
N_OUTER = B_PER_W // K_INNER
DRAIN_ROWS = 128

_mesh = plsc.VectorSubcoreMesh(core_axis_name="c", subcore_axis_name="s")


@functools.partial(
    pl.kernel,
    mesh=_mesh,
    out_type=jax.ShapeDtypeStruct((B_TOTAL, HIDDEN), jnp.float32),
    scratch_types=[
        pltpu.VMEM((B_PER_W,), jnp.int32),
        pltpu.VMEM((DRAIN_ROWS, HIDDEN), jnp.float32),
        pltpu.SemaphoreType.DMA,
    ],
)
def _gather_rows(idx_hbm, table_hbm, out_hbm, idx_s, drain_v, sem):
    wid = lax.axis_index("s") * NUM_CORES + lax.axis_index("c")
    base = wid * B_PER_W
    pltpu.sync_copy(idx_hbm.at[pl.ds(base, B_PER_W)], idx_s)

    def body(g, _):
        idx_vec = idx_s[pl.ds(g * K_INNER, K_INNER)]
        for j in range(K_INNER):
            row = idx_vec[j]
            pltpu.async_copy(
                table_hbm.at[pl.ds(row, 1)],
                out_hbm.at[pl.ds(base + g * K_INNER + j, 1)],
                sem).wait()
        return ()

    lax.fori_loop(0, N_OUTER, body, (), unroll=False)


def kernel(position_ids, table):
    idx = position_ids.reshape(-1)
    out = _gather_rows(idx, table)
    return out.reshape(position_ids.shape + (HIDDEN,))


# speedup vs baseline: 33.3676x; 33.3676x over previous
import functools

import jax
import jax.numpy as jnp
from jax import lax
from jax.experimental import pallas as pl
from jax.experimental.pallas import tpu as pltpu
from jax.experimental.pallas import tpu_sc as plsc

HIDDEN = 768
NUM_CORES = 2
NUM_SUBCORES = 16
NUM_WORKERS = NUM_CORES * NUM_SUBCORES

B_TOTAL = 4 * 8192
B_PER_W = B_TOTAL // NUM_WORKERS
CHUNK = 32
N_CHUNKS = B_PER_W // CHUNK
NBUF = 4

_mesh = plsc.VectorSubcoreMesh(core_axis_name="c", subcore_axis_name="s")


@functools.partial(
    pl.kernel,
    mesh=_mesh,
    out_type=jax.ShapeDtypeStruct((B_TOTAL, HIDDEN), jnp.float32),
    scratch_types=(
        [pltpu.VMEM((B_PER_W,), jnp.int32)]
        + [pltpu.VMEM((CHUNK, HIDDEN), jnp.float32) for _ in range(NBUF)]
        + [pltpu.SemaphoreType.DMA for _ in range(2 * NBUF)]
    ),
)
def _gather_rows(idx_hbm, table_hbm, out_hbm, idx_v, *scratch):
    bufs = scratch[:NBUF]
    gsems = scratch[NBUF:2 * NBUF]
    osems = scratch[2 * NBUF:]
    wid = lax.axis_index("s") * NUM_CORES + lax.axis_index("c")
    base = wid * B_PER_W
    pltpu.sync_copy(idx_hbm.at[pl.ds(base, B_PER_W)], idx_v)

    def start_gather(c, slot):
        return pltpu.async_copy(
            table_hbm.at[idx_v.at[pl.ds(c * CHUNK, CHUNK)]],
            bufs[slot], gsems[slot])

    gather_pending = [None] * NBUF
    out_pending = [None] * NBUF
    for slot in range(NBUF - 1):
        gather_pending[slot] = start_gather(slot, slot)
    for c in range(N_CHUNKS):
        slot = c % NBUF
        nxt = (c + NBUF - 1) % NBUF
        if c + NBUF - 1 < N_CHUNKS:
            if out_pending[nxt] is not None:
                out_pending[nxt].wait()
                out_pending[nxt] = None
            gather_pending[nxt] = start_gather(c + NBUF - 1, nxt)
        gather_pending[slot].wait()
        out_pending[slot] = pltpu.async_copy(
            bufs[slot], out_hbm.at[pl.ds(base + c * CHUNK, CHUNK)],
            osems[slot])
    for slot in range(NBUF):
        if out_pending[slot] is not None:
            out_pending[slot].wait()


def kernel(position_ids, table):
    idx = position_ids.reshape(-1)
    out = _gather_rows(idx, table)
    return out.reshape(position_ids.shape + (HIDDEN,))


# speedup vs baseline: 33.5500x; 1.0055x over previous
import functools

import jax
import jax.numpy as jnp
from jax import lax
from jax.experimental import pallas as pl
from jax.experimental.pallas import tpu as pltpu
from jax.experimental.pallas import tpu_sc as plsc

HIDDEN = 768
NUM_CORES = 2
NUM_SUBCORES = 16
NUM_WORKERS = NUM_CORES * NUM_SUBCORES

B_TOTAL = 4 * 8192
B_PER_W = B_TOTAL // NUM_WORKERS
CHUNK = 32
N_CHUNKS = B_PER_W // CHUNK
NBUF = 5

_mesh = plsc.VectorSubcoreMesh(core_axis_name="c", subcore_axis_name="s")


@functools.partial(
    pl.kernel,
    mesh=_mesh,
    out_type=jax.ShapeDtypeStruct((B_TOTAL, HIDDEN), jnp.float32),
    scratch_types=(
        [pltpu.VMEM((B_PER_W,), jnp.int32)]
        + [pltpu.VMEM((CHUNK, HIDDEN), jnp.float32) for _ in range(NBUF)]
        + [pltpu.SemaphoreType.DMA for _ in range(2 * NBUF)]
    ),
)
def _gather_rows(idx_hbm, table_hbm, out_hbm, idx_v, *scratch):
    bufs = scratch[:NBUF]
    gsems = scratch[NBUF:2 * NBUF]
    osems = scratch[2 * NBUF:]
    wid = lax.axis_index("s") * NUM_CORES + lax.axis_index("c")
    base = wid * B_PER_W
    pltpu.sync_copy(idx_hbm.at[pl.ds(base, CHUNK)], idx_v.at[pl.ds(0, CHUNK)])

    def start_gather(c, slot):
        return pltpu.async_copy(
            table_hbm.at[idx_v.at[pl.ds(c * CHUNK, CHUNK)]],
            bufs[slot], gsems[slot])

    gather_pending = [None] * NBUF
    out_pending = [None] * NBUF
    gather_pending[0] = start_gather(0, 0)
    pltpu.sync_copy(idx_hbm.at[pl.ds(base + CHUNK, B_PER_W - CHUNK)],
                    idx_v.at[pl.ds(CHUNK, B_PER_W - CHUNK)])
    for slot in range(1, NBUF - 1):
        gather_pending[slot] = start_gather(slot, slot)
    for c in range(N_CHUNKS):
        slot = c % NBUF
        nxt = (c + NBUF - 1) % NBUF
        if c + NBUF - 1 < N_CHUNKS:
            if out_pending[nxt] is not None:
                out_pending[nxt].wait()
                out_pending[nxt] = None
            gather_pending[nxt] = start_gather(c + NBUF - 1, nxt)
        gather_pending[slot].wait()
        out_pending[slot] = pltpu.async_copy(
            bufs[slot], out_hbm.at[pl.ds(base + c * CHUNK, CHUNK)],
            osems[slot])
    for slot in range(NBUF):
        if out_pending[slot] is not None:
            out_pending[slot].wait()


def kernel(position_ids, table):
    idx = position_ids.reshape(-1)
    out = _gather_rows(idx, table)
    return out.reshape(position_ids.shape + (HIDDEN,))
